# baseline (speedup 1.0000x reference)
import functools

import jax
import jax.numpy as jnp
from jax.experimental import pallas as pl
from jax.experimental.pallas import tpu as pltpu

LANE = 128
SUBLANE = 8

_A1 = 3.141592502593994
_A3 = -5.1677069664001465
_A5 = 2.5500314235687256
_A7 = -0.5980454683303833
_A9 = 0.07722075283527374


def _fast_sin_pi_units(t):
    kf = jax.lax.round(t, jax.lax.RoundingMethod.TO_NEAREST_EVEN)
    ki = kf.astype(jnp.int32)
    r = t - kf
    r2 = r * r
    s = r * (_A1 + r2 * (_A3 + r2 * (_A5 + r2 * (_A7 + r2 * _A9))))
    sign = jnp.left_shift(jnp.bitwise_and(ki, 1), 31)
    bits = jax.lax.bitcast_convert_type(s, jnp.int32) ^ sign
    return jax.lax.bitcast_convert_type(bits, jnp.float32)


def _round_up(n: int, m: int) -> int:
    return ((n + m - 1) // m) * m


def _sin_mlp_kernel(x_ref, w1_ref, b1_ref, w2_ref, w3_ref, b23_ref, o_ref):
    x = x_ref[...].astype(jnp.bfloat16)
    t = jnp.dot(x, w1_ref[...], preferred_element_type=jnp.float32)
    s = _fast_sin_pi_units(t + b1_ref[...])
    z = jnp.dot(s, w2_ref[...], preferred_element_type=jnp.float32)
    r = w3_ref.shape[0]
    y = b23_ref[...] + z[:, 0:1] * w3_ref[0:1, :]
    for j in range(1, r):
        y = y + z[:, j : j + 1] * w3_ref[j : j + 1, :]
    o_ref[...] = y


@functools.partial(jax.jit, static_argnames=("tb",))
def _forward(x, w1, b1, w2, b2, w3, b3, *, tb=2048):
    B, d_in = x.shape
    H = w1.shape[1]
    r, d_out = w3.shape
    n_pad = _round_up(d_out, LANE)

    inv_pi = 1.0 / jnp.pi
    w1b = (w1 * inv_pi).astype(jnp.bfloat16)
    w2p = jnp.zeros((H, LANE), jnp.float32).at[:, :r].set(w2)
    b23 = (jnp.dot(b2, w3, preferred_element_type=jnp.float32) + b3).reshape(1, -1)
    if n_pad != d_out:
        w3 = jnp.pad(w3, ((0, 0), (0, n_pad - d_out)))
        b23 = jnp.pad(b23, ((0, 0), (0, n_pad - d_out)))
    b1r = (b1 * inv_pi).reshape(1, -1)

    TB = min(tb, _round_up(B, SUBLANE))
    B_pad = _round_up(B, TB)
    if B_pad != B:
        x = jnp.pad(x, ((0, B_pad - B), (0, 0)))

    out = pl.pallas_call(
        _sin_mlp_kernel,
        out_shape=jax.ShapeDtypeStruct((B_pad, n_pad), jnp.float32),
        grid=(B_pad // TB,),
        in_specs=[
            pl.BlockSpec((TB, d_in), lambda i: (i, 0)),
            pl.BlockSpec((d_in, H), lambda i: (0, 0)),
            pl.BlockSpec((1, H), lambda i: (0, 0)),
            pl.BlockSpec((H, LANE), lambda i: (0, 0)),
            pl.BlockSpec((r, n_pad), lambda i: (0, 0)),
            pl.BlockSpec((1, n_pad), lambda i: (0, 0)),
        ],
        out_specs=pl.BlockSpec((TB, n_pad), lambda i: (i, 0)),
        compiler_params=pltpu.CompilerParams(
            dimension_semantics=("parallel",),
        ),
    )(x, w1b, b1r, w2p, w3, b23)

    if B_pad != B or n_pad != d_out:
        out = out[:B, :d_out]
    return out


def kernel(x, w1, b1, w2, b2, w3, b3):
    return _forward(x, w1, b1, w2, b2, w3, b3)


# speedup vs baseline: 1.2491x; 1.2491x over previous
import functools

import jax
import jax.numpy as jnp
from jax.experimental import pallas as pl
from jax.experimental.pallas import tpu as pltpu

LANE = 128
SUBLANE = 8

_A1 = 3.1406409740448
_A3 = -5.1369242668151855
_A5 = 2.299621343612671


def _fast_sin_pi_units(t):
    kf = jax.lax.round(t, jax.lax.RoundingMethod.TO_NEAREST_EVEN)
    ki = kf.astype(jnp.int32)
    r = t - kf
    r2 = r * r
    s = r * (_A1 + r2 * (_A3 + r2 * _A5))
    sign = jnp.left_shift(jnp.bitwise_and(ki, 1), 31)
    bits = jax.lax.bitcast_convert_type(s, jnp.int32) ^ sign
    return jax.lax.bitcast_convert_type(bits, jnp.float32)


def _round_up(n: int, m: int) -> int:
    return ((n + m - 1) // m) * m


def _sin_mlp_kernel(x_ref, w1_ref, b1_ref, w2_ref, w3_ref, b23_ref, o_ref):
    x = x_ref[...].astype(jnp.bfloat16)
    t = jnp.dot(x, w1_ref[...], preferred_element_type=jnp.float32)
    s = _fast_sin_pi_units(t + b1_ref[...])
    z = jnp.dot(s, w2_ref[...], preferred_element_type=jnp.float32)
    r = w3_ref.shape[0]
    y = b23_ref[...] + z[:, 0:1] * w3_ref[0:1, :]
    for j in range(1, r):
        y = y + z[:, j : j + 1] * w3_ref[j : j + 1, :]
    o_ref[...] = y


@functools.partial(jax.jit, static_argnames=("tb",))
def _forward(x, w1, b1, w2, b2, w3, b3, *, tb=1024):
    B, d_in = x.shape
    H = w1.shape[1]
    r, d_out = w3.shape
    n_pad = _round_up(d_out, LANE)

    inv_pi = 1.0 / jnp.pi
    w1b = (w1 * inv_pi).astype(jnp.bfloat16)
    w2p = jnp.zeros((H, LANE), jnp.float32).at[:, :r].set(w2)
    b23 = (jnp.dot(b2, w3, preferred_element_type=jnp.float32) + b3).reshape(1, -1)
    if n_pad != d_out:
        w3 = jnp.pad(w3, ((0, 0), (0, n_pad - d_out)))
        b23 = jnp.pad(b23, ((0, 0), (0, n_pad - d_out)))
    b1r = (b1 * inv_pi).reshape(1, -1)

    TB = min(tb, _round_up(B, SUBLANE))
    B_pad = _round_up(B, TB)
    if B_pad != B:
        x = jnp.pad(x, ((0, B_pad - B), (0, 0)))

    out = pl.pallas_call(
        _sin_mlp_kernel,
        out_shape=jax.ShapeDtypeStruct((B_pad, n_pad), jnp.float32),
        grid=(B_pad // TB,),
        in_specs=[
            pl.BlockSpec((TB, d_in), lambda i: (i, 0)),
            pl.BlockSpec((d_in, H), lambda i: (0, 0)),
            pl.BlockSpec((1, H), lambda i: (0, 0)),
            pl.BlockSpec((H, LANE), lambda i: (0, 0)),
            pl.BlockSpec((r, n_pad), lambda i: (0, 0)),
            pl.BlockSpec((1, n_pad), lambda i: (0, 0)),
        ],
        out_specs=pl.BlockSpec((TB, n_pad), lambda i: (i, 0)),
        compiler_params=pltpu.CompilerParams(
            dimension_semantics=("parallel",),
        ),
    )(x, w1b, b1r, w2p, w3, b23)

    if B_pad != B or n_pad != d_out:
        out = out[:B, :d_out]
    return out


def kernel(x, w1, b1, w2, b2, w3, b3):
    return _forward(x, w1, b1, w2, b2, w3, b3)


# speedup vs baseline: 1.3366x; 1.0701x over previous
import functools

import jax
import jax.numpy as jnp
from jax.experimental import pallas as pl
from jax.experimental.pallas import tpu as pltpu

LANE = 128
SUBLANE = 8

_INV_PI = 0.3183098861837907
_A1 = 3.1406409740448
_A3 = -5.1369242668151855
_A5 = 2.299621343612671


def _fast_sin_pi_units(t):
    kf = jax.lax.round(t, jax.lax.RoundingMethod.TO_NEAREST_EVEN)
    ki = kf.astype(jnp.int32)
    r = t - kf
    r2 = r * r
    s = r * (_A1 + r2 * (_A3 + r2 * _A5))
    sign = jnp.left_shift(jnp.bitwise_and(ki, 1), 31)
    bits = jax.lax.bitcast_convert_type(s, jnp.int32) ^ sign
    return jax.lax.bitcast_convert_type(bits, jnp.float32)


def _round_up(n: int, m: int) -> int:
    return ((n + m - 1) // m) * m


def _sin_mlp_kernel(
    x_ref, w1_ref, b1_ref, w2_ref, b2_ref, w3_ref, b3_ref, o_ref,
    w1s_ref, b1s_ref, b23_ref,
):
    @pl.when(pl.program_id(0) == 0)
    def _prep():
        w1s_ref[...] = (w1_ref[...] * _INV_PI).astype(jnp.bfloat16)
        b1s_ref[...] = b1_ref[...] * _INV_PI
        b23_ref[...] = (
            b3_ref[...]
            + b2_ref[0:1, 0:1] * w3_ref[0:1, :]
            + b2_ref[0:1, 1:2] * w3_ref[1:2, :]
        )

    x = x_ref[...].astype(jnp.bfloat16)
    t = jnp.dot(x, w1s_ref[...], preferred_element_type=jnp.float32)
    s = _fast_sin_pi_units(t + b1s_ref[...])
    z = jnp.dot(s, w2_ref[...], preferred_element_type=jnp.float32)
    y = (
        b23_ref[...]
        + z[:, 0:1] * w3_ref[0:1, :]
        + z[:, 1:2] * w3_ref[1:2, :]
    )
    o_ref[...] = y


@functools.partial(jax.jit, static_argnames=("tb",))
def _forward(x, w1, b1, w2, b2, w3, b3, *, tb=1024):
    B, d_in = x.shape
    H = w1.shape[1]
    r, d_out = w3.shape

    b1r = b1.reshape(1, -1)
    b2r = b2.reshape(1, -1)
    b3r = b3.reshape(1, -1)

    TB = min(tb, _round_up(B, SUBLANE))
    B_pad = _round_up(B, TB)
    if B_pad != B:
        x = jnp.pad(x, ((0, B_pad - B), (0, 0)))

    out = pl.pallas_call(
        _sin_mlp_kernel,
        out_shape=jax.ShapeDtypeStruct((B_pad, d_out), jnp.float32),
        grid=(B_pad // TB,),
        in_specs=[
            pl.BlockSpec((TB, d_in), lambda i: (i, 0)),
            pl.BlockSpec((d_in, H), lambda i: (0, 0)),
            pl.BlockSpec((1, H), lambda i: (0, 0)),
            pl.BlockSpec((H, r), lambda i: (0, 0)),
            pl.BlockSpec((1, r), lambda i: (0, 0)),
            pl.BlockSpec((r, d_out), lambda i: (0, 0)),
            pl.BlockSpec((1, d_out), lambda i: (0, 0)),
        ],
        out_specs=pl.BlockSpec((TB, d_out), lambda i: (i, 0)),
        scratch_shapes=[
            pltpu.VMEM((d_in, H), jnp.bfloat16),
            pltpu.VMEM((1, H), jnp.float32),
            pltpu.VMEM((1, d_out), jnp.float32),
        ],
        compiler_params=pltpu.CompilerParams(
            dimension_semantics=("arbitrary",),
        ),
    )(x, w1, b1r, w2, b2r, w3, b3r)

    if B_pad != B:
        out = out[:B]
    return out


def kernel(x, w1, b1, w2, b2, w3, b3):
    return _forward(x, w1, b1, w2, b2, w3, b3)
